# baseline (speedup 1.0000x reference)
import math
from functools import partial

import jax
import jax.numpy as jnp
from jax.experimental import pallas as pl
from jax.experimental.pallas import tpu as pltpu

_H = 16
_FACTOR = 5
_UPAD = 48


def _qkv_matmul_kernel(x_ref, wq_ref, wk_ref, wv_ref, bq_ref, bk_ref, bv_ref,
                       q_ref, k_ref, v_ref):
    kk = pl.program_id(2)

    @pl.when(kk == 0)
    def _init():
        q_ref[:, :] = jnp.broadcast_to(bq_ref[:, :], q_ref.shape)
        k_ref[:, :] = jnp.broadcast_to(bk_ref[:, :], k_ref.shape)
        v_ref[:, :] = jnp.broadcast_to(bv_ref[:, :], v_ref.shape)

    xb = x_ref[:, :]
    q_ref[:, :] += jnp.dot(xb, wq_ref[:, :], preferred_element_type=jnp.float32)
    k_ref[:, :] += jnp.dot(xb, wk_ref[:, :], preferred_element_type=jnp.float32)
    v_ref[:, :] += jnp.dot(xb, wv_ref[:, :], preferred_element_type=jnp.float32)


def _ks_gather_kernel(samp_ref, k_ref, ks_ref, *, U):
    for u in range(U):
        ks_ref[pl.ds(u, 1), :] = k_ref[pl.ds(samp_ref[u], 1), :]
    for u in range(U, _UPAD):
        ks_ref[pl.ds(u, 1), :] = jnp.zeros((1, k_ref.shape[1]), jnp.float32)


def _scores_kernel(q_ref, ks_ref, m_ref, *, U, dk):
    for h in range(_H):
        s = jax.lax.dot_general(
            q_ref[:, pl.ds(h * dk, dk)], ks_ref[:, pl.ds(h * dk, dk)],
            (((1,), (1,)), ((), ())),
            preferred_element_type=jnp.float32,
        )
        col = jax.lax.broadcasted_iota(jnp.int32, s.shape, 1)
        valid = col < U
        smax = jnp.max(jnp.where(valid, s, -jnp.inf), axis=1)
        smean = jnp.sum(jnp.where(valid, s, 0.0), axis=1) / float(U)
        m_ref[h, 0, :] = smax - smean


def _topk_kernel(m_ref, idx_ref, *, U, L):
    m = m_ref[:, 0, :]
    rows = m.shape[0]
    l_iota = jax.lax.broadcasted_iota(jnp.int32, (rows, L), 1)
    c_iota = jax.lax.broadcasted_iota(jnp.int32, (rows, 128), 1)
    acc = jnp.zeros((rows, 128), jnp.int32)
    for u in range(U):
        cur = jnp.max(m, axis=1, keepdims=True)
        ismax = m == cur
        idx = jnp.min(jnp.where(ismax, l_iota, L), axis=1)
        acc = jnp.where(c_iota == u, idx[:, None], acc)
        m = jnp.where(l_iota == idx[:, None], -jnp.inf, m)
    idx_ref[:, :] = acc


def _qgather_kernel(idx_ref, q_ref, qs_ref, *, U, dk):
    b = pl.program_id(0)
    zrow = jnp.zeros((1, dk), jnp.float32)
    for h in range(_H):
        for u in range(U):
            i = idx_ref[b * _H + h, u]
            row = q_ref[pl.ds(i, 1), :]
            qs_ref[pl.ds(h * _UPAD + u, 1), :] = row[:, h * dk:(h + 1) * dk]
        for u in range(U, _UPAD):
            qs_ref[pl.ds(h * _UPAD + u, 1), :] = zrow


def _attn_kernel(qs_ref, k_ref, v_ref, wo_ref, corr_ref, mv_ref, *, L, dk):
    qs = qs_ref[:, :]
    s = jax.lax.dot_general(
        qs, k_ref[:, :], (((1,), (1,)), ((), ())),
        preferred_element_type=jnp.float32,
    ) * (1.0 / math.sqrt(dk))
    smax = jnp.max(s, axis=1, keepdims=True)
    e = jnp.exp(s - smax)
    p = e / jnp.sum(e, axis=1, keepdims=True)
    o = jnp.dot(p, v_ref[:, :], preferred_element_type=jnp.float32)
    mv = jnp.sum(v_ref[:, :], axis=0, keepdims=True) * (1.0 / L)
    mv_ref[0, :, :] = mv
    corr_ref[:, :] = jnp.dot(
        o - mv, wo_ref[:, :], preferred_element_type=jnp.float32
    )


def _assemble_kernel(idx_ref, corr_ref, mv_ref, wo_ref, bo_ref, out_ref,
                     *, U, dk):
    b = pl.program_id(0)
    base = bo_ref[:, :]
    for h in range(_H):
        base = base + jnp.dot(
            mv_ref[h, :, :], wo_ref[pl.ds(h * dk, dk), :],
            preferred_element_type=jnp.float32,
        )
    out_ref[:, :] = jnp.broadcast_to(base, out_ref.shape)
    for h in range(_H):
        def body(u, carry, h=h):
            r = idx_ref[b * _H + h, u]
            out_ref[pl.ds(r, 1), :] = (
                out_ref[pl.ds(r, 1), :] + corr_ref[pl.ds(h * _UPAD + u, 1), :]
            )
            return carry
        jax.lax.fori_loop(0, U, body, 0)


def kernel(x, Wq, bq, Wk, bk, Wv, bv, Wo, bo):
    B, L, D = x.shape
    dk = D // _H
    U = max(1, int(_FACTOR * math.log(L + 1)))
    U = min(U, L)
    samp = jax.random.randint(jax.random.key(42), (U,), 0, L).astype(jnp.int32)

    x2d = x.reshape(B * L, D)

    bm, bn, bk_ = 2048, 512, 512
    q2d, k2d, v2d = pl.pallas_call(
        _qkv_matmul_kernel,
        grid=(B * L // bm, D // bn, D // bk_),
        in_specs=[
            pl.BlockSpec((bm, bk_), lambda i, j, kk: (i, kk)),
            pl.BlockSpec((bk_, bn), lambda i, j, kk: (kk, j)),
            pl.BlockSpec((bk_, bn), lambda i, j, kk: (kk, j)),
            pl.BlockSpec((bk_, bn), lambda i, j, kk: (kk, j)),
            pl.BlockSpec((1, bn), lambda i, j, kk: (0, j)),
            pl.BlockSpec((1, bn), lambda i, j, kk: (0, j)),
            pl.BlockSpec((1, bn), lambda i, j, kk: (0, j)),
        ],
        out_specs=[
            pl.BlockSpec((bm, bn), lambda i, j, kk: (i, j)),
            pl.BlockSpec((bm, bn), lambda i, j, kk: (i, j)),
            pl.BlockSpec((bm, bn), lambda i, j, kk: (i, j)),
        ],
        out_shape=[
            jax.ShapeDtypeStruct((B * L, D), jnp.float32),
            jax.ShapeDtypeStruct((B * L, D), jnp.float32),
            jax.ShapeDtypeStruct((B * L, D), jnp.float32),
        ],
        compiler_params=pltpu.CompilerParams(
            dimension_semantics=("parallel", "parallel", "arbitrary"),
        ),
    )(x2d, Wq, Wk, Wv, bq[None, :], bk[None, :], bv[None, :])

    ks = pl.pallas_call(
        partial(_ks_gather_kernel, U=U),
        grid_spec=pltpu.PrefetchScalarGridSpec(
            num_scalar_prefetch=1,
            grid=(B,),
            in_specs=[pl.BlockSpec((L, D), lambda b, samp: (b, 0))],
            out_specs=pl.BlockSpec((_UPAD, D), lambda b, samp: (b, 0)),
        ),
        out_shape=jax.ShapeDtypeStruct((B * _UPAD, D), jnp.float32),
    )(samp, k2d)

    m = pl.pallas_call(
        partial(_scores_kernel, U=U, dk=dk),
        grid=(B,),
        in_specs=[
            pl.BlockSpec((L, D), lambda b: (b, 0)),
            pl.BlockSpec((_UPAD, D), lambda b: (b, 0)),
        ],
        out_specs=pl.BlockSpec((_H, 1, L), lambda b: (b, 0, 0)),
        out_shape=jax.ShapeDtypeStruct((B * _H, 1, L), jnp.float32),
        compiler_params=pltpu.CompilerParams(
            dimension_semantics=("arbitrary",),
        ),
    )(q2d, ks)

    top_idx = pl.pallas_call(
        partial(_topk_kernel, U=U, L=L),
        in_specs=[pl.BlockSpec((B * _H, 1, L), lambda: (0, 0, 0))],
        out_specs=pl.BlockSpec((B * _H, 128), lambda: (0, 0)),
        out_shape=jax.ShapeDtypeStruct((B * _H, 128), jnp.int32),
    )(m)

    qs = pl.pallas_call(
        partial(_qgather_kernel, U=U, dk=dk),
        grid_spec=pltpu.PrefetchScalarGridSpec(
            num_scalar_prefetch=1,
            grid=(B,),
            in_specs=[pl.BlockSpec((L, D), lambda b, idx: (b, 0))],
            out_specs=pl.BlockSpec((_H * _UPAD, dk), lambda b, idx: (b, 0)),
        ),
        out_shape=jax.ShapeDtypeStruct((B * _H * _UPAD, dk), jnp.float32),
        compiler_params=pltpu.CompilerParams(
            dimension_semantics=("arbitrary",),
        ),
    )(top_idx, q2d)

    corr, mv = pl.pallas_call(
        partial(_attn_kernel, L=L, dk=dk),
        grid=(B, _H),
        in_specs=[
            pl.BlockSpec((_UPAD, dk), lambda b, h: (b * _H + h, 0)),
            pl.BlockSpec((L, dk), lambda b, h: (b, h)),
            pl.BlockSpec((L, dk), lambda b, h: (b, h)),
            pl.BlockSpec((dk, D), lambda b, h: (h, 0)),
        ],
        out_specs=[
            pl.BlockSpec((_UPAD, D), lambda b, h: (b * _H + h, 0)),
            pl.BlockSpec((1, 1, dk), lambda b, h: (b * _H + h, 0, 0)),
        ],
        out_shape=[
            jax.ShapeDtypeStruct((B * _H * _UPAD, D), jnp.float32),
            jax.ShapeDtypeStruct((B * _H, 1, dk), jnp.float32),
        ],
        compiler_params=pltpu.CompilerParams(
            dimension_semantics=("parallel", "parallel"),
        ),
    )(qs, k2d, v2d, Wo)

    ncol = 2
    bcol = D // ncol
    out2d = pl.pallas_call(
        partial(_assemble_kernel, U=U, dk=dk),
        grid_spec=pltpu.PrefetchScalarGridSpec(
            num_scalar_prefetch=1,
            grid=(B, ncol),
            in_specs=[
                pl.BlockSpec((_H * _UPAD, bcol), lambda b, j, idx: (b, j)),
                pl.BlockSpec((_H, 1, dk), lambda b, j, idx: (b, 0, 0)),
                pl.BlockSpec((D, bcol), lambda b, j, idx: (0, j)),
                pl.BlockSpec((1, bcol), lambda b, j, idx: (0, j)),
            ],
            out_specs=pl.BlockSpec((L, bcol), lambda b, j, idx: (b, j)),
        ),
        out_shape=jax.ShapeDtypeStruct((B * L, D), jnp.float32),
        compiler_params=pltpu.CompilerParams(
            dimension_semantics=("arbitrary", "arbitrary"),
        ),
    )(top_idx, corr, mv, Wo, bo[None, :])

    return out2d.reshape(B, L, D)


# speedup vs baseline: 1.0745x; 1.0745x over previous
import math
from functools import partial

import jax
import jax.numpy as jnp
from jax.experimental import pallas as pl
from jax.experimental.pallas import tpu as pltpu

_H = 16
_FACTOR = 5
_UPAD = 48


def _qkv_matmul_kernel(x_ref, wq_ref, wk_ref, wv_ref, bq_ref, bk_ref, bv_ref,
                       q_ref, k_ref, v_ref):
    xb = x_ref[:, :]
    q_ref[:, :] = (
        jnp.dot(xb, wq_ref[:, :], preferred_element_type=jnp.float32)
        + bq_ref[:, :]
    )
    k_ref[:, :] = (
        jnp.dot(xb, wk_ref[:, :], preferred_element_type=jnp.float32)
        + bk_ref[:, :]
    )
    v_ref[:, :] = (
        jnp.dot(xb, wv_ref[:, :], preferred_element_type=jnp.float32)
        + bv_ref[:, :]
    )


def _ks_gather_kernel(samp_ref, k_ref, ks_ref, *, U):
    for u in range(U):
        ks_ref[pl.ds(u, 1), :] = k_ref[pl.ds(samp_ref[u], 1), :]
    for u in range(U, _UPAD):
        ks_ref[pl.ds(u, 1), :] = jnp.zeros((1, k_ref.shape[1]), jnp.float32)


def _scores_kernel(q_ref, ks_ref, m_ref, *, U, dk):
    for h in range(_H):
        s = jax.lax.dot_general(
            q_ref[:, pl.ds(h * dk, dk)], ks_ref[:, pl.ds(h * dk, dk)],
            (((1,), (1,)), ((), ())),
            preferred_element_type=jnp.float32,
        )
        col = jax.lax.broadcasted_iota(jnp.int32, s.shape, 1)
        valid = col < U
        smax = jnp.max(jnp.where(valid, s, -jnp.inf), axis=1)
        smean = jnp.sum(jnp.where(valid, s, 0.0), axis=1) / float(U)
        m_ref[h, 0, :] = smax - smean


def _topk_kernel(m_ref, idx_ref, *, U, L):
    m = m_ref[:, 0, :]
    rows = m.shape[0]
    l_iota = jax.lax.broadcasted_iota(jnp.int32, (rows, L), 1)
    c_iota = jax.lax.broadcasted_iota(jnp.int32, (rows, 128), 1)
    acc = jnp.zeros((rows, 128), jnp.int32)
    for u in range(U):
        cur = jnp.max(m, axis=1, keepdims=True)
        ismax = m == cur
        idx = jnp.min(jnp.where(ismax, l_iota, L), axis=1)
        acc = jnp.where(c_iota == u, idx[:, None], acc)
        m = jnp.where(l_iota == idx[:, None], -jnp.inf, m)
    idx_ref[:, :] = acc


def _qgather_kernel(idx_ref, q_ref, qs_ref, *, U, dk):
    b = pl.program_id(0)
    zrow = jnp.zeros((1, dk), jnp.float32)
    for h in range(_H):
        for u in range(U):
            i = idx_ref[b * _H + h, u]
            row = q_ref[pl.ds(i, 1), :]
            qs_ref[pl.ds(h * _UPAD + u, 1), :] = row[:, h * dk:(h + 1) * dk]
        for u in range(U, _UPAD):
            qs_ref[pl.ds(h * _UPAD + u, 1), :] = zrow


def _attn_kernel(qs_ref, k_ref, v_ref, wo_ref, corr_ref, mv_ref, *, L, dk):
    qs = qs_ref[:, :]
    s = jax.lax.dot_general(
        qs, k_ref[:, :], (((1,), (1,)), ((), ())),
        preferred_element_type=jnp.float32,
    ) * (1.0 / math.sqrt(dk))
    smax = jnp.max(s, axis=1, keepdims=True)
    e = jnp.exp(s - smax)
    p = e / jnp.sum(e, axis=1, keepdims=True)
    o = jnp.dot(p, v_ref[:, :], preferred_element_type=jnp.float32)
    mv = jnp.sum(v_ref[:, :], axis=0, keepdims=True) * (1.0 / L)
    mv_ref[0, :, :] = mv
    corr_ref[:, :] = jnp.dot(
        o - mv, wo_ref[:, :], preferred_element_type=jnp.float32
    )


def _assemble_kernel(idx_ref, corr_ref, mv_ref, wo_ref, bo_ref, out_ref,
                     *, U, dk):
    b = pl.program_id(0)
    base = bo_ref[:, :]
    for h in range(_H):
        base = base + jnp.dot(
            mv_ref[h, :, :], wo_ref[pl.ds(h * dk, dk), :],
            preferred_element_type=jnp.float32,
        )
    out_ref[:, :] = jnp.broadcast_to(base, out_ref.shape)
    for h in range(_H):
        def body(u, carry, h=h):
            r = idx_ref[b * _H + h, u]
            out_ref[pl.ds(r, 1), :] = (
                out_ref[pl.ds(r, 1), :] + corr_ref[pl.ds(h * _UPAD + u, 1), :]
            )
            return carry
        jax.lax.fori_loop(0, U, body, 0)


def kernel(x, Wq, bq, Wk, bk, Wv, bv, Wo, bo):
    B, L, D = x.shape
    dk = D // _H
    U = max(1, int(_FACTOR * math.log(L + 1)))
    U = min(U, L)
    samp = jax.random.randint(jax.random.key(42), (U,), 0, L).astype(jnp.int32)

    x2d = x.reshape(B * L, D)

    bm, bn = 1024, 512
    q2d, k2d, v2d = pl.pallas_call(
        _qkv_matmul_kernel,
        grid=(D // bn, B * L // bm),
        in_specs=[
            pl.BlockSpec((bm, D), lambda j, i: (i, 0)),
            pl.BlockSpec((D, bn), lambda j, i: (0, j)),
            pl.BlockSpec((D, bn), lambda j, i: (0, j)),
            pl.BlockSpec((D, bn), lambda j, i: (0, j)),
            pl.BlockSpec((1, bn), lambda j, i: (0, j)),
            pl.BlockSpec((1, bn), lambda j, i: (0, j)),
            pl.BlockSpec((1, bn), lambda j, i: (0, j)),
        ],
        out_specs=[
            pl.BlockSpec((bm, bn), lambda j, i: (i, j)),
            pl.BlockSpec((bm, bn), lambda j, i: (i, j)),
            pl.BlockSpec((bm, bn), lambda j, i: (i, j)),
        ],
        out_shape=[
            jax.ShapeDtypeStruct((B * L, D), jnp.float32),
            jax.ShapeDtypeStruct((B * L, D), jnp.float32),
            jax.ShapeDtypeStruct((B * L, D), jnp.float32),
        ],
        compiler_params=pltpu.CompilerParams(
            dimension_semantics=("parallel", "parallel"),
        ),
    )(x2d, Wq, Wk, Wv, bq[None, :], bk[None, :], bv[None, :])

    ks = pl.pallas_call(
        partial(_ks_gather_kernel, U=U),
        grid_spec=pltpu.PrefetchScalarGridSpec(
            num_scalar_prefetch=1,
            grid=(B,),
            in_specs=[pl.BlockSpec((L, D), lambda b, samp: (b, 0))],
            out_specs=pl.BlockSpec((_UPAD, D), lambda b, samp: (b, 0)),
        ),
        out_shape=jax.ShapeDtypeStruct((B * _UPAD, D), jnp.float32),
    )(samp, k2d)

    m = pl.pallas_call(
        partial(_scores_kernel, U=U, dk=dk),
        grid=(B,),
        in_specs=[
            pl.BlockSpec((L, D), lambda b: (b, 0)),
            pl.BlockSpec((_UPAD, D), lambda b: (b, 0)),
        ],
        out_specs=pl.BlockSpec((_H, 1, L), lambda b: (b, 0, 0)),
        out_shape=jax.ShapeDtypeStruct((B * _H, 1, L), jnp.float32),
        compiler_params=pltpu.CompilerParams(
            dimension_semantics=("arbitrary",),
        ),
    )(q2d, ks)

    top_idx = pl.pallas_call(
        partial(_topk_kernel, U=U, L=L),
        in_specs=[pl.BlockSpec((B * _H, 1, L), lambda: (0, 0, 0))],
        out_specs=pl.BlockSpec((B * _H, 128), lambda: (0, 0)),
        out_shape=jax.ShapeDtypeStruct((B * _H, 128), jnp.int32),
    )(m)

    qs = pl.pallas_call(
        partial(_qgather_kernel, U=U, dk=dk),
        grid_spec=pltpu.PrefetchScalarGridSpec(
            num_scalar_prefetch=1,
            grid=(B,),
            in_specs=[pl.BlockSpec((L, D), lambda b, idx: (b, 0))],
            out_specs=pl.BlockSpec((_H * _UPAD, dk), lambda b, idx: (b, 0)),
        ),
        out_shape=jax.ShapeDtypeStruct((B * _H * _UPAD, dk), jnp.float32),
        compiler_params=pltpu.CompilerParams(
            dimension_semantics=("arbitrary",),
        ),
    )(top_idx, q2d)

    corr, mv = pl.pallas_call(
        partial(_attn_kernel, L=L, dk=dk),
        grid=(B, _H),
        in_specs=[
            pl.BlockSpec((_UPAD, dk), lambda b, h: (b * _H + h, 0)),
            pl.BlockSpec((L, dk), lambda b, h: (b, h)),
            pl.BlockSpec((L, dk), lambda b, h: (b, h)),
            pl.BlockSpec((dk, D), lambda b, h: (h, 0)),
        ],
        out_specs=[
            pl.BlockSpec((_UPAD, D), lambda b, h: (b * _H + h, 0)),
            pl.BlockSpec((1, 1, dk), lambda b, h: (b * _H + h, 0, 0)),
        ],
        out_shape=[
            jax.ShapeDtypeStruct((B * _H * _UPAD, D), jnp.float32),
            jax.ShapeDtypeStruct((B * _H, 1, dk), jnp.float32),
        ],
        compiler_params=pltpu.CompilerParams(
            dimension_semantics=("parallel", "parallel"),
        ),
    )(qs, k2d, v2d, Wo)

    ncol = 2
    bcol = D // ncol
    out2d = pl.pallas_call(
        partial(_assemble_kernel, U=U, dk=dk),
        grid_spec=pltpu.PrefetchScalarGridSpec(
            num_scalar_prefetch=1,
            grid=(B, ncol),
            in_specs=[
                pl.BlockSpec((_H * _UPAD, bcol), lambda b, j, idx: (b, j)),
                pl.BlockSpec((_H, 1, dk), lambda b, j, idx: (b, 0, 0)),
                pl.BlockSpec((D, bcol), lambda b, j, idx: (0, j)),
                pl.BlockSpec((1, bcol), lambda b, j, idx: (0, j)),
            ],
            out_specs=pl.BlockSpec((L, bcol), lambda b, j, idx: (b, j)),
        ),
        out_shape=jax.ShapeDtypeStruct((B * L, D), jnp.float32),
        compiler_params=pltpu.CompilerParams(
            dimension_semantics=("arbitrary", "arbitrary"),
        ),
    )(top_idx, corr, mv, Wo, bo[None, :])

    return out2d.reshape(B, L, D)


# speedup vs baseline: 1.0823x; 1.0073x over previous
import math
from functools import partial

import jax
import jax.numpy as jnp
from jax.experimental import pallas as pl
from jax.experimental.pallas import tpu as pltpu

_H = 16
_FACTOR = 5
_UPAD = 48


def _qkv_matmul_kernel(x_ref, wq_ref, wk_ref, wv_ref, bq_ref, bk_ref, bv_ref,
                       q_ref, k_ref, v_ref):
    xb = x_ref[:, :]
    q_ref[:, :] = (
        jnp.dot(xb, wq_ref[:, :], preferred_element_type=jnp.float32)
        + bq_ref[:, :]
    )
    k_ref[:, :] = (
        jnp.dot(xb, wk_ref[:, :], preferred_element_type=jnp.float32)
        + bk_ref[:, :]
    )
    v_ref[:, :] = (
        jnp.dot(xb, wv_ref[:, :], preferred_element_type=jnp.float32)
        + bv_ref[:, :]
    )


def _ks_gather_kernel(samp_ref, k_ref, ks_ref, *, U):
    for u in range(U):
        ks_ref[pl.ds(u, 1), :] = k_ref[pl.ds(samp_ref[u], 1), :]
    for u in range(U, _UPAD):
        ks_ref[pl.ds(u, 1), :] = jnp.zeros((1, k_ref.shape[1]), jnp.float32)


def _scores_kernel(q_ref, ks_ref, m_ref, *, U, dk):
    for h in range(_H):
        s = jax.lax.dot_general(
            q_ref[:, pl.ds(h * dk, dk)], ks_ref[:, pl.ds(h * dk, dk)],
            (((1,), (1,)), ((), ())),
            preferred_element_type=jnp.float32,
        )
        col = jax.lax.broadcasted_iota(jnp.int32, s.shape, 1)
        valid = col < U
        smax = jnp.max(jnp.where(valid, s, -jnp.inf), axis=1)
        smean = jnp.sum(jnp.where(valid, s, 0.0), axis=1) / float(U)
        m_ref[h, 0, :] = smax - smean


def _topk_kernel(m_ref, idx_ref, *, U, L):
    m = m_ref[:, 0, :]
    rows = m.shape[0]
    l_iota = jax.lax.broadcasted_iota(jnp.int32, (rows, L), 1)
    c_iota = jax.lax.broadcasted_iota(jnp.int32, (rows, 128), 1)
    acc = jnp.zeros((rows, 128), jnp.int32)
    for u in range(U):
        cur = jnp.max(m, axis=1, keepdims=True)
        ismax = m == cur
        idx = jnp.min(jnp.where(ismax, l_iota, L), axis=1)
        acc = jnp.where(c_iota == u, idx[:, None], acc)
        m = jnp.where(l_iota == idx[:, None], -jnp.inf, m)
    idx_ref[:, :] = acc


def _qgather_kernel(idx_ref, q_ref, qs_ref, *, U, dk):
    b = pl.program_id(0)
    zrow = jnp.zeros((1, dk), jnp.float32)
    for h in range(_H):
        for u in range(U):
            i = idx_ref[b * _H + h, u]
            row = q_ref[pl.ds(i, 1), :]
            qs_ref[pl.ds(h * _UPAD + u, 1), :] = row[:, h * dk:(h + 1) * dk]
        for u in range(U, _UPAD):
            qs_ref[pl.ds(h * _UPAD + u, 1), :] = zrow


def _attn_kernel(qs_ref, k_ref, v_ref, wo_ref, corr_ref, mv_ref, *, L, dk):
    qs = qs_ref[:, :]
    s = jax.lax.dot_general(
        qs, k_ref[:, :], (((1,), (1,)), ((), ())),
        preferred_element_type=jnp.float32,
    ) * (1.0 / math.sqrt(dk))
    smax = jnp.max(s, axis=1, keepdims=True)
    e = jnp.exp(s - smax)
    p = e / jnp.sum(e, axis=1, keepdims=True)
    o = jnp.dot(p, v_ref[:, :], preferred_element_type=jnp.float32)
    mv = jnp.sum(v_ref[:, :], axis=0, keepdims=True) * (1.0 / L)
    mv_ref[0, :, :] = mv
    corr_ref[:, :] = jnp.dot(
        o - mv, wo_ref[:, :], preferred_element_type=jnp.float32
    )


def _assemble_kernel(idx_ref, corr_ref, mv_ref, wo_ref, bo_ref, out_ref,
                     *, U, dk):
    b = pl.program_id(0)
    base = bo_ref[:, :]
    for h in range(_H):
        base = base + jnp.dot(
            mv_ref[h, :, :], wo_ref[pl.ds(h * dk, dk), :],
            preferred_element_type=jnp.float32,
        )
    out_ref[:, :] = jnp.broadcast_to(base, out_ref.shape)
    for h in range(_H):
        def body(u, carry, h=h):
            r = idx_ref[b * _H + h, u]
            out_ref[pl.ds(r, 1), :] = (
                out_ref[pl.ds(r, 1), :] + corr_ref[pl.ds(h * _UPAD + u, 1), :]
            )
            return carry
        jax.lax.fori_loop(0, U, body, 0)


def kernel(x, Wq, bq, Wk, bk, Wv, bv, Wo, bo):
    B, L, D = x.shape
    dk = D // _H
    U = max(1, int(_FACTOR * math.log(L + 1)))
    U = min(U, L)
    samp = jax.random.randint(jax.random.key(42), (U,), 0, L).astype(jnp.int32)

    x2d = x.reshape(B * L, D)

    bm, bn = 1024, 512
    q2d, k2d, v2d = pl.pallas_call(
        _qkv_matmul_kernel,
        grid=(D // bn, B * L // bm),
        in_specs=[
            pl.BlockSpec((bm, D), lambda j, i: (i, 0)),
            pl.BlockSpec((D, bn), lambda j, i: (0, j)),
            pl.BlockSpec((D, bn), lambda j, i: (0, j)),
            pl.BlockSpec((D, bn), lambda j, i: (0, j)),
            pl.BlockSpec((1, bn), lambda j, i: (0, j)),
            pl.BlockSpec((1, bn), lambda j, i: (0, j)),
            pl.BlockSpec((1, bn), lambda j, i: (0, j)),
        ],
        out_specs=[
            pl.BlockSpec((bm, bn), lambda j, i: (i, j)),
            pl.BlockSpec((bm, bn), lambda j, i: (i, j)),
            pl.BlockSpec((bm, bn), lambda j, i: (i, j)),
        ],
        out_shape=[
            jax.ShapeDtypeStruct((B * L, D), jnp.float32),
            jax.ShapeDtypeStruct((B * L, D), jnp.float32),
            jax.ShapeDtypeStruct((B * L, D), jnp.float32),
        ],
        compiler_params=pltpu.CompilerParams(
            dimension_semantics=("parallel", "parallel"),
        ),
    )(x2d, Wq, Wk, Wv, bq[None, :], bk[None, :], bv[None, :])

    ks = pl.pallas_call(
        partial(_ks_gather_kernel, U=U),
        grid_spec=pltpu.PrefetchScalarGridSpec(
            num_scalar_prefetch=1,
            grid=(B,),
            in_specs=[pl.BlockSpec((L, D), lambda b, samp: (b, 0))],
            out_specs=pl.BlockSpec((_UPAD, D), lambda b, samp: (b, 0)),
        ),
        out_shape=jax.ShapeDtypeStruct((B * _UPAD, D), jnp.float32),
    )(samp, k2d)

    m = pl.pallas_call(
        partial(_scores_kernel, U=U, dk=dk),
        grid=(B,),
        in_specs=[
            pl.BlockSpec((L, D), lambda b: (b, 0)),
            pl.BlockSpec((_UPAD, D), lambda b: (b, 0)),
        ],
        out_specs=pl.BlockSpec((_H, 1, L), lambda b: (b, 0, 0)),
        out_shape=jax.ShapeDtypeStruct((B * _H, 1, L), jnp.float32),
        compiler_params=pltpu.CompilerParams(
            dimension_semantics=("arbitrary",),
        ),
    )(q2d, ks)

    top_idx = pl.pallas_call(
        partial(_topk_kernel, U=U, L=L),
        in_specs=[pl.BlockSpec((B * _H, 1, L), lambda: (0, 0, 0))],
        out_specs=pl.BlockSpec((B * _H, 128), lambda: (0, 0)),
        out_shape=jax.ShapeDtypeStruct((B * _H, 128), jnp.int32),
    )(m)

    qs = pl.pallas_call(
        partial(_qgather_kernel, U=U, dk=dk),
        grid_spec=pltpu.PrefetchScalarGridSpec(
            num_scalar_prefetch=1,
            grid=(B,),
            in_specs=[pl.BlockSpec((L, D), lambda b, idx: (b, 0))],
            out_specs=pl.BlockSpec((_H * _UPAD, dk), lambda b, idx: (b, 0)),
        ),
        out_shape=jax.ShapeDtypeStruct((B * _H * _UPAD, dk), jnp.float32),
        compiler_params=pltpu.CompilerParams(
            dimension_semantics=("arbitrary",),
        ),
    )(top_idx, q2d)

    corr, mv = pl.pallas_call(
        partial(_attn_kernel, L=L, dk=dk),
        grid=(_H, B),
        in_specs=[
            pl.BlockSpec((_UPAD, dk), lambda h, b: (b * _H + h, 0)),
            pl.BlockSpec((L, dk), lambda h, b: (b, h)),
            pl.BlockSpec((L, dk), lambda h, b: (b, h)),
            pl.BlockSpec((dk, D), lambda h, b: (h, 0)),
        ],
        out_specs=[
            pl.BlockSpec((_UPAD, D), lambda h, b: (b * _H + h, 0)),
            pl.BlockSpec((1, 1, dk), lambda h, b: (b * _H + h, 0, 0)),
        ],
        out_shape=[
            jax.ShapeDtypeStruct((B * _H * _UPAD, D), jnp.float32),
            jax.ShapeDtypeStruct((B * _H, 1, dk), jnp.float32),
        ],
        compiler_params=pltpu.CompilerParams(
            dimension_semantics=("parallel", "parallel"),
        ),
    )(qs, k2d, v2d, Wo)

    ncol = 2
    bcol = D // ncol
    out2d = pl.pallas_call(
        partial(_assemble_kernel, U=U, dk=dk),
        grid_spec=pltpu.PrefetchScalarGridSpec(
            num_scalar_prefetch=1,
            grid=(ncol, B),
            in_specs=[
                pl.BlockSpec((_H * _UPAD, bcol), lambda j, b, idx: (b, j)),
                pl.BlockSpec((_H, 1, dk), lambda j, b, idx: (b, 0, 0)),
                pl.BlockSpec((D, bcol), lambda j, b, idx: (0, j)),
                pl.BlockSpec((1, bcol), lambda j, b, idx: (0, j)),
            ],
            out_specs=pl.BlockSpec((L, bcol), lambda j, b, idx: (b, j)),
        ),
        out_shape=jax.ShapeDtypeStruct((B * L, D), jnp.float32),
        compiler_params=pltpu.CompilerParams(
            dimension_semantics=("arbitrary", "arbitrary"),
        ),
    )(top_idx, corr, mv, Wo, bo[None, :])

    return out2d.reshape(B, L, D)


# speedup vs baseline: 1.0834x; 1.0010x over previous
import math
from functools import partial

import jax
import jax.numpy as jnp
from jax.experimental import pallas as pl
from jax.experimental.pallas import tpu as pltpu

_H = 16
_FACTOR = 5
_UPAD = 48


def _qkv_matmul_kernel(x_ref, wq_ref, wk_ref, wv_ref, bq_ref, bk_ref, bv_ref,
                       q_ref, k_ref, v_ref):
    xb = x_ref[:, :]
    q_ref[:, :] = (
        jnp.dot(xb, wq_ref[:, :], preferred_element_type=jnp.float32)
        + bq_ref[:, :]
    )
    k_ref[:, :] = (
        jnp.dot(xb, wk_ref[:, :], preferred_element_type=jnp.float32)
        + bk_ref[:, :]
    )
    v_ref[:, :] = (
        jnp.dot(xb, wv_ref[:, :], preferred_element_type=jnp.float32)
        + bv_ref[:, :]
    )


def _ks_gather_kernel(samp_ref, k_ref, ks_ref, *, U):
    for u in range(U):
        ks_ref[pl.ds(u, 1), :] = k_ref[pl.ds(samp_ref[u], 1), :]
    for u in range(U, _UPAD):
        ks_ref[pl.ds(u, 1), :] = jnp.zeros((1, k_ref.shape[1]), jnp.float32)


def _scores_kernel(q_ref, ks_ref, m_ref, *, U, dk):
    for h in range(_H):
        s = jax.lax.dot_general(
            q_ref[:, pl.ds(h * dk, dk)], ks_ref[:, pl.ds(h * dk, dk)],
            (((1,), (1,)), ((), ())),
            preferred_element_type=jnp.float32,
        )
        col = jax.lax.broadcasted_iota(jnp.int32, s.shape, 1)
        valid = col < U
        smax = jnp.max(jnp.where(valid, s, -jnp.inf), axis=1)
        smean = jnp.sum(jnp.where(valid, s, 0.0), axis=1) / float(U)
        m_ref[h, 0, :] = smax - smean


def _topk_kernel(m_ref, idx_ref, *, U, L):
    m = m_ref[:, 0, :]
    rows = m.shape[0]
    l_iota = jax.lax.broadcasted_iota(jnp.int32, (rows, L), 1)
    c_iota = jax.lax.broadcasted_iota(jnp.int32, (rows, 128), 1)
    acc = jnp.zeros((rows, 128), jnp.int32)
    for u in range(U):
        cur = jnp.max(m, axis=1, keepdims=True)
        ismax = m == cur
        idx = jnp.min(jnp.where(ismax, l_iota, L), axis=1)
        acc = jnp.where(c_iota == u, idx[:, None], acc)
        m = jnp.where(l_iota == idx[:, None], -jnp.inf, m)
    idx_ref[:, :] = acc


def _qgather_kernel(idx_ref, q_ref, qs_ref, *, U, dk):
    b = pl.program_id(0)
    zrow = jnp.zeros((1, dk), jnp.float32)
    for h in range(_H):
        for u in range(U):
            i = idx_ref[b * _H + h, u]
            row = q_ref[pl.ds(i, 1), :]
            qs_ref[pl.ds(h * _UPAD + u, 1), :] = row[:, h * dk:(h + 1) * dk]
        for u in range(U, _UPAD):
            qs_ref[pl.ds(h * _UPAD + u, 1), :] = zrow


def _attn_kernel(qs_ref, k_ref, v_ref, wo_ref, corr_ref, mv_ref, *, L, dk):
    qs = qs_ref[:, :]
    s = jax.lax.dot_general(
        qs, k_ref[:, :], (((1,), (1,)), ((), ())),
        preferred_element_type=jnp.float32,
    ) * (1.0 / math.sqrt(dk))
    smax = jnp.max(s, axis=1, keepdims=True)
    e = jnp.exp(s - smax)
    p = e / jnp.sum(e, axis=1, keepdims=True)
    o = jnp.dot(p, v_ref[:, :], preferred_element_type=jnp.float32)
    mv = jnp.sum(v_ref[:, :], axis=0, keepdims=True) * (1.0 / L)
    mv_ref[0, :, :] = mv
    corr_ref[:, :] = jnp.dot(
        o - mv, wo_ref[:, :], preferred_element_type=jnp.float32
    )


def _assemble_kernel(idx_ref, corr_ref, mv_ref, wo_ref, bo_ref, out_ref,
                     *, U, dk):
    b = pl.program_id(1)
    base = bo_ref[:, :]
    for h in range(_H):
        base = base + jnp.dot(
            mv_ref[h, :, :], wo_ref[pl.ds(h * dk, dk), :],
            preferred_element_type=jnp.float32,
        )
    out_ref[:, :] = jnp.broadcast_to(base, out_ref.shape)
    for h in range(_H):
        def body(u, carry, h=h):
            r = idx_ref[b * _H + h, u]
            out_ref[pl.ds(r, 1), :] = (
                out_ref[pl.ds(r, 1), :] + corr_ref[pl.ds(h * _UPAD + u, 1), :]
            )
            return carry
        jax.lax.fori_loop(0, U, body, 0)


def kernel(x, Wq, bq, Wk, bk, Wv, bv, Wo, bo):
    B, L, D = x.shape
    dk = D // _H
    U = max(1, int(_FACTOR * math.log(L + 1)))
    U = min(U, L)
    samp = jax.random.randint(jax.random.key(42), (U,), 0, L).astype(jnp.int32)

    x2d = x.reshape(B * L, D)

    bm, bn = 1024, 512
    q2d, k2d, v2d = pl.pallas_call(
        _qkv_matmul_kernel,
        grid=(D // bn, B * L // bm),
        in_specs=[
            pl.BlockSpec((bm, D), lambda j, i: (i, 0)),
            pl.BlockSpec((D, bn), lambda j, i: (0, j)),
            pl.BlockSpec((D, bn), lambda j, i: (0, j)),
            pl.BlockSpec((D, bn), lambda j, i: (0, j)),
            pl.BlockSpec((1, bn), lambda j, i: (0, j)),
            pl.BlockSpec((1, bn), lambda j, i: (0, j)),
            pl.BlockSpec((1, bn), lambda j, i: (0, j)),
        ],
        out_specs=[
            pl.BlockSpec((bm, bn), lambda j, i: (i, j)),
            pl.BlockSpec((bm, bn), lambda j, i: (i, j)),
            pl.BlockSpec((bm, bn), lambda j, i: (i, j)),
        ],
        out_shape=[
            jax.ShapeDtypeStruct((B * L, D), jnp.float32),
            jax.ShapeDtypeStruct((B * L, D), jnp.float32),
            jax.ShapeDtypeStruct((B * L, D), jnp.float32),
        ],
        compiler_params=pltpu.CompilerParams(
            dimension_semantics=("parallel", "parallel"),
        ),
    )(x2d, Wq, Wk, Wv, bq[None, :], bk[None, :], bv[None, :])

    ks = pl.pallas_call(
        partial(_ks_gather_kernel, U=U),
        grid_spec=pltpu.PrefetchScalarGridSpec(
            num_scalar_prefetch=1,
            grid=(B,),
            in_specs=[pl.BlockSpec((L, D), lambda b, samp: (b, 0))],
            out_specs=pl.BlockSpec((_UPAD, D), lambda b, samp: (b, 0)),
        ),
        out_shape=jax.ShapeDtypeStruct((B * _UPAD, D), jnp.float32),
    )(samp, k2d)

    m = pl.pallas_call(
        partial(_scores_kernel, U=U, dk=dk),
        grid=(B,),
        in_specs=[
            pl.BlockSpec((L, D), lambda b: (b, 0)),
            pl.BlockSpec((_UPAD, D), lambda b: (b, 0)),
        ],
        out_specs=pl.BlockSpec((_H, 1, L), lambda b: (b, 0, 0)),
        out_shape=jax.ShapeDtypeStruct((B * _H, 1, L), jnp.float32),
        compiler_params=pltpu.CompilerParams(
            dimension_semantics=("arbitrary",),
        ),
    )(q2d, ks)

    top_idx = pl.pallas_call(
        partial(_topk_kernel, U=U, L=L),
        in_specs=[pl.BlockSpec((B * _H, 1, L), lambda: (0, 0, 0))],
        out_specs=pl.BlockSpec((B * _H, 128), lambda: (0, 0)),
        out_shape=jax.ShapeDtypeStruct((B * _H, 128), jnp.int32),
    )(m)

    qs = pl.pallas_call(
        partial(_qgather_kernel, U=U, dk=dk),
        grid_spec=pltpu.PrefetchScalarGridSpec(
            num_scalar_prefetch=1,
            grid=(B,),
            in_specs=[pl.BlockSpec((L, D), lambda b, idx: (b, 0))],
            out_specs=pl.BlockSpec((_H * _UPAD, dk), lambda b, idx: (b, 0)),
        ),
        out_shape=jax.ShapeDtypeStruct((B * _H * _UPAD, dk), jnp.float32),
        compiler_params=pltpu.CompilerParams(
            dimension_semantics=("arbitrary",),
        ),
    )(top_idx, q2d)

    corr, mv = pl.pallas_call(
        partial(_attn_kernel, L=L, dk=dk),
        grid=(_H, B),
        in_specs=[
            pl.BlockSpec((_UPAD, dk), lambda h, b: (b * _H + h, 0)),
            pl.BlockSpec((L, dk), lambda h, b: (b, h)),
            pl.BlockSpec((L, dk), lambda h, b: (b, h)),
            pl.BlockSpec((dk, D), lambda h, b: (h, 0)),
        ],
        out_specs=[
            pl.BlockSpec((_UPAD, D), lambda h, b: (b * _H + h, 0)),
            pl.BlockSpec((1, 1, dk), lambda h, b: (b * _H + h, 0, 0)),
        ],
        out_shape=[
            jax.ShapeDtypeStruct((B * _H * _UPAD, D), jnp.float32),
            jax.ShapeDtypeStruct((B * _H, 1, dk), jnp.float32),
        ],
        compiler_params=pltpu.CompilerParams(
            dimension_semantics=("parallel", "parallel"),
        ),
    )(qs, k2d, v2d, Wo)

    ncol = 2
    bcol = D // ncol
    out2d = pl.pallas_call(
        partial(_assemble_kernel, U=U, dk=dk),
        grid_spec=pltpu.PrefetchScalarGridSpec(
            num_scalar_prefetch=1,
            grid=(ncol, B),
            in_specs=[
                pl.BlockSpec((_H * _UPAD, bcol), lambda j, b, idx: (b, j)),
                pl.BlockSpec((_H, 1, dk), lambda j, b, idx: (b, 0, 0)),
                pl.BlockSpec((D, bcol), lambda j, b, idx: (0, j)),
                pl.BlockSpec((1, bcol), lambda j, b, idx: (0, j)),
            ],
            out_specs=pl.BlockSpec((L, bcol), lambda j, b, idx: (b, j)),
        ),
        out_shape=jax.ShapeDtypeStruct((B * L, D), jnp.float32),
        compiler_params=pltpu.CompilerParams(
            dimension_semantics=("arbitrary", "arbitrary"),
        ),
    )(top_idx, corr, mv, Wo, bo[None, :])

    return out2d.reshape(B, L, D)
